# Initial kernel scaffold; baseline (speedup 1.0000x reference)
import jax
import jax.numpy as jnp
from jax.experimental import pallas as pl


def kernel(x, batch, edge_index, cross_edge_index, inner_edge_index, c1_W1, c1_b1, c1_bn_g, c1_bn_b, c1_W2, c1_b2, i1_W1, i1_b1, i1_bn_g, i1_bn_b, i1_W2, i1_b2, lin1_W, lin1_b, lin2_W, lin2_b):
    raise NotImplementedError("write your pallas kernel here")


# speedup vs baseline: 27.7476x; 27.7476x over previous
import jax
import jax.numpy as jnp
from jax.experimental import pallas as pl

B, N0, N1, IN, D, OUT = 64, 1024, 256, 64, 64, 10
E0, EC, EI = 16384, 1024, 4096
NB = N0 * B
NBC = N1 * B
F = B * D

_EA_CH = 2048


def _builder_body(ei_ref, ce_ref, ie_ref, a_ref, p_ref, a2_ref):
    c = pl.program_id(0)

    def onehot_pair(src, dst, nsrc, ndst, e):
        ohd = (jax.lax.broadcasted_iota(jnp.int32, (ndst, e), 0) == dst
               ).astype(jnp.bfloat16)
        ohs = (jax.lax.broadcasted_iota(jnp.int32, (nsrc, e), 0) == src
               ).astype(jnp.bfloat16)
        return jax.lax.dot_general(ohd, ohs, (((1,), (1,)), ((), ())),
                                   preferred_element_type=jnp.float32)

    @pl.when(c == 0)
    def _small():
        p_ref[...] = onehot_pair(ce_ref[0:1, :], ce_ref[1:2, :], N0, N1, EC)
        a2_ref[...] = onehot_pair(ie_ref[0:1, :], ie_ref[1:2, :], N1, N1, EI)

    src = ei_ref[0:1, pl.ds(c * _EA_CH, _EA_CH)]
    dst = ei_ref[1:2, pl.ds(c * _EA_CH, _EA_CH)]
    contrib = onehot_pair(src, dst, N0, N0, _EA_CH)

    @pl.when(c == 0)
    def _init():
        a_ref[...] = contrib

    @pl.when(c > 0)
    def _acc():
        a_ref[...] += contrib


def _build_mats(ei, ce, ie):
    return pl.pallas_call(
        _builder_body,
        grid=(E0 // _EA_CH,),
        in_specs=[
            pl.BlockSpec((2, E0), lambda c: (0, 0)),
            pl.BlockSpec((2, EC), lambda c: (0, 0)),
            pl.BlockSpec((2, EI), lambda c: (0, 0)),
        ],
        out_specs=[
            pl.BlockSpec((N0, N0), lambda c: (0, 0)),
            pl.BlockSpec((N1, N0), lambda c: (0, 0)),
            pl.BlockSpec((N1, N1), lambda c: (0, 0)),
        ],
        out_shape=[
            jax.ShapeDtypeStruct((N0, N0), jnp.float32),
            jax.ShapeDtypeStruct((N1, N0), jnp.float32),
            jax.ShapeDtypeStruct((N1, N1), jnp.float32),
        ],
    )(ei, ce, ie)


def _lin_body(x_ref, w_ref, o_ref):
    o_ref[...] = jax.lax.dot_general(
        x_ref[...], w_ref[...], (((1,), (1,)), ((), ())),
        preferred_element_type=jnp.float32)


def _lin(x, w, mblk):
    m = x.shape[0]
    return pl.pallas_call(
        _lin_body,
        grid=(m // mblk,),
        in_specs=[
            pl.BlockSpec((mblk, x.shape[1]), lambda i: (i, 0)),
            pl.BlockSpec(w.shape, lambda i: (0, 0)),
        ],
        out_specs=pl.BlockSpec((mblk, w.shape[0]), lambda i: (i, 0)),
        out_shape=jax.ShapeDtypeStruct((m, w.shape[0]), jnp.float32),
    )(x, w)


def _mix_body(z_ref, a_ref, b_ref, h_ref, st_ref):
    z = z_ref[...]
    h = z + jnp.dot(a_ref[...], z, preferred_element_type=jnp.float32)
    h = h + b_ref[...]
    h_ref[...] = h
    st_ref[0:1, :] = jnp.sum(h, axis=0, keepdims=True)
    st_ref[1:2, :] = jnp.sum(h * h, axis=0, keepdims=True)


def _mix(zv, a, bias_t, nblk):
    n = zv.shape[0]
    return pl.pallas_call(
        _mix_body,
        grid=(F // nblk,),
        in_specs=[
            pl.BlockSpec((n, nblk), lambda j: (0, j)),
            pl.BlockSpec((n, n), lambda j: (0, 0)),
            pl.BlockSpec((1, nblk), lambda j: (0, j)),
        ],
        out_specs=[
            pl.BlockSpec((n, nblk), lambda j: (0, j)),
            pl.BlockSpec((2, nblk), lambda j: (0, j)),
        ],
        out_shape=[
            jax.ShapeDtypeStruct((n, F), jnp.float32),
            jax.ShapeDtypeStruct((2, F), jnp.float32),
        ],
    )(zv, a, bias_t)


def _bnlin_body(h_ref, ssum_ref, ssq_ref, g_ref, bb_ref, w2_ref, b2_ref,
                o_ref, *, nrows):
    inv = 1.0 / nrows
    mean = jnp.sum(ssum_ref[...], axis=0, keepdims=True) * inv
    ex2 = jnp.sum(ssq_ref[...], axis=0, keepdims=True) * inv
    var = ex2 - mean * mean
    scale = g_ref[...] * jax.lax.rsqrt(var + 1e-5)
    shift = bb_ref[...] - mean * scale
    hb = jnp.maximum(h_ref[...] * scale + shift, 0.0)
    o = jax.lax.dot_general(hb, w2_ref[...], (((1,), (1,)), ((), ())),
                            preferred_element_type=jnp.float32)
    o_ref[...] = jnp.maximum(o + b2_ref[...], 0.0)


def _bnlin(h, ssum, ssq, g, bb, w2, b2, mblk):
    import functools
    m = h.shape[0]
    body = functools.partial(_bnlin_body, nrows=m)
    return pl.pallas_call(
        body,
        grid=(m // mblk,),
        in_specs=[
            pl.BlockSpec((mblk, D), lambda i: (i, 0)),
            pl.BlockSpec((B, D), lambda i: (0, 0)),
            pl.BlockSpec((B, D), lambda i: (0, 0)),
            pl.BlockSpec((1, D), lambda i: (0, 0)),
            pl.BlockSpec((1, D), lambda i: (0, 0)),
            pl.BlockSpec((D, D), lambda i: (0, 0)),
            pl.BlockSpec((1, D), lambda i: (0, 0)),
        ],
        out_specs=pl.BlockSpec((mblk, D), lambda i: (i, 0)),
        out_shape=jax.ShapeDtypeStruct((m, D), jnp.float32),
    )(h, ssum, ssq, g, bb, w2, b2)


def _pool_body(h_ref, p_ref, o_ref):
    pfull = p_ref[...]
    cnt = jnp.sum(pfull, axis=1, keepdims=True)
    recip = 1.0 / jnp.maximum(cnt, 1.0)
    s = jnp.dot(pfull, h_ref[...], preferred_element_type=jnp.float32)
    o_ref[...] = s * recip


def _pool(h2v, pm, nblk):
    return pl.pallas_call(
        _pool_body,
        grid=(F // nblk,),
        in_specs=[
            pl.BlockSpec((N0, nblk), lambda j: (0, j)),
            pl.BlockSpec((N1, N0), lambda j: (0, 0)),
        ],
        out_specs=pl.BlockSpec((N1, nblk), lambda j: (0, j)),
        out_shape=jax.ShapeDtypeStruct((N1, F), jnp.float32),
    )(h2v, pm)


def _readout_body(h_ref, w1_ref, b1_ref, w2_ref, b2_ref, o_ref):
    t = jax.lax.dot_general(h_ref[...], w1_ref[...], (((1,), (1,)), ((), ())),
                            preferred_element_type=jnp.float32)
    t = jnp.maximum(t + b1_ref[...], 0.0)
    o = jax.lax.dot_general(t, w2_ref[...], (((1,), (1,)), ((), ())),
                            preferred_element_type=jnp.float32)
    o_ref[...] = o + b2_ref[...]


def _readout(hbm, w1, b1, w2, b2):
    return pl.pallas_call(
        _readout_body,
        in_specs=[
            pl.BlockSpec((B, N1 * D), lambda: (0, 0)),
            pl.BlockSpec((D, N1 * D), lambda: (0, 0)),
            pl.BlockSpec((1, D), lambda: (0, 0)),
            pl.BlockSpec((OUT, D), lambda: (0, 0)),
            pl.BlockSpec((1, OUT), lambda: (0, 0)),
        ],
        out_specs=pl.BlockSpec((B, OUT), lambda: (0, 0)),
        out_shape=jax.ShapeDtypeStruct((B, OUT), jnp.float32),
    )(hbm, w1, b1, w2, b2)


def kernel(x, batch, edge_index, cross_edge_index, inner_edge_index,
           c1_W1, c1_b1, c1_bn_g, c1_bn_b, c1_W2, c1_b2,
           i1_W1, i1_b1, i1_bn_g, i1_bn_b, i1_W2, i1_b2,
           lin1_W, lin1_b, lin2_W, lin2_b):
    del batch
    a, pm, a2 = _build_mats(edge_index, cross_edge_index, inner_edge_index)

    x2 = x.reshape(B, N0, IN).transpose(1, 0, 2).reshape(NB, IN)

    z = _lin(x2, c1_W1, 4096)
    h1, st1 = _mix(z.reshape(N0, F), a, jnp.tile(c1_b1, B)[None], 512)
    h2 = _bnlin(h1.reshape(NB, D), st1[0].reshape(B, D), st1[1].reshape(B, D),
                c1_bn_g[None], c1_bn_b[None], c1_W2, c1_b2[None], 4096)
    hp = _pool(h2.reshape(N0, F), pm, 512)
    zp = _lin(hp.reshape(NBC, D), i1_W1, 4096)
    g1, st2 = _mix(zp.reshape(N1, F), a2, jnp.tile(i1_b1, B)[None], 512)
    h3 = _bnlin(g1.reshape(NBC, D), st2[0].reshape(B, D), st2[1].reshape(B, D),
                i1_bn_g[None], i1_bn_b[None], i1_W2, i1_b2[None], 4096)

    h3bm = h3.reshape(N1, B, D).transpose(1, 0, 2).reshape(B, N1 * D)
    return _readout(h3bm, lin1_W, lin1_b[None], lin2_W, lin2_b[None])
